# Initial kernel scaffold; baseline (speedup 1.0000x reference)
import jax
import jax.numpy as jnp
from jax.experimental import pallas as pl


def kernel(x, edge_index, W1, b1, g1, bt1, W2, b2, g2, bt2, W3, b3):
    raise NotImplementedError("write your pallas kernel here")


# speedup vs baseline: 9.0753x; 9.0753x over previous
import functools

import jax
import jax.numpy as jnp
from jax import lax
from jax.experimental import pallas as pl
from jax.experimental.pallas import tpu as pltpu
from jax.experimental.pallas import tpu_sc as plsc

NC = 2
NS = 16
L = 16
K = 128
ZR = 64
BR = 2000


def _mesh():
    return plsc.VectorSubcoreMesh(core_axis_name="c", subcore_axis_name="s")


@functools.cache
def _agg_kernel(tab_rows, et, n_acc, f):
    nb = et // K
    assert et % (2 * K) == 0 and n_acc % (NS * ZR) == 0

    @functools.partial(
        pl.kernel,
        out_type=jax.ShapeDtypeStruct((NC * n_acc, f), jnp.float32),
        mesh=_mesh(),
        scratch_types=[
            pltpu.VMEM((K,), jnp.int32),
            pltpu.VMEM((K,), jnp.int32),
            pltpu.VMEM((K,), jnp.int32),
            pltpu.VMEM((K,), jnp.int32),
            pltpu.VMEM((K, f), jnp.float32),
            pltpu.VMEM((K, f), jnp.float32),
            pltpu.VMEM((ZR, f), jnp.float32),
            pltpu.VMEM_SHARED((n_acc, f), jnp.float32),
            pltpu.SemaphoreType.DMA,
            pltpu.SemaphoreType.DMA,
        ],
    )
    def agg(table, src, dst, zrows, out,
            src_a, src_b, dst_a, dst_b, rows_a, rows_b, zbuf, acc,
            sem_a, sem_b):
        c = lax.axis_index("c")
        s = lax.axis_index("s")
        rpt = n_acc // NS
        nz = rpt // ZR

        pltpu.sync_copy(zrows, zbuf)

        @pl.loop(0, nz)
        def _zero(i):
            r0 = pl.multiple_of(s * rpt + i * ZR, ZR)
            pltpu.sync_copy(zbuf, acc.at[pl.ds(r0, ZR)])

        plsc.subcore_barrier()

        ebase = (c * NS + s) * et

        pltpu.sync_copy(src.at[pl.ds(pl.multiple_of(ebase, K), K)], src_a)
        pltpu.sync_copy(dst.at[pl.ds(pl.multiple_of(ebase, K), K)], dst_a)
        pltpu.async_copy(table.at[src_a], rows_a, sem_a)

        @pl.loop(0, nb, step=2)
        def _edges(i):
            o1 = pl.multiple_of(ebase + (i + 1) * K, K)
            pltpu.sync_copy(src.at[pl.ds(o1, K)], src_b)
            pltpu.sync_copy(dst.at[pl.ds(o1, K)], dst_b)
            pltpu.make_async_copy(table.at[src_a], rows_a, sem_a).wait()
            pltpu.async_copy(table.at[src_b], rows_b, sem_b)
            pltpu.sync_copy(rows_a, acc.at[dst_a], add=True)

            @pl.when(i + 2 < nb)
            def _prefetch():
                o2 = pl.multiple_of(ebase + (i + 2) * K, K)
                pltpu.sync_copy(src.at[pl.ds(o2, K)], src_a)
                pltpu.sync_copy(dst.at[pl.ds(o2, K)], dst_a)
                pltpu.async_copy(table.at[src_a], rows_a, sem_a)

            pltpu.make_async_copy(table.at[src_b], rows_b, sem_b).wait()
            pltpu.sync_copy(rows_b, acc.at[dst_b], add=True)

        plsc.subcore_barrier()

        @pl.loop(0, nz)
        def _drain(i):
            r0 = pl.multiple_of(s * rpt + i * ZR, ZR)
            pltpu.sync_copy(acc.at[pl.ds(r0, ZR)], zbuf)
            pltpu.sync_copy(zbuf, out.at[pl.ds(c * n_acc + r0, ZR)])

    return agg


@functools.cache
def _deg_kernel(et, nh):
    nb = et // K
    seg = nh // NS
    assert nh % (NS * L) == 0

    @functools.partial(
        pl.kernel,
        out_type=jax.ShapeDtypeStruct((NC * nh,), jnp.float32),
        mesh=_mesh(),
        compiler_params=pltpu.CompilerParams(needs_layout_passes=False),
        scratch_types=[
            pltpu.VMEM((nh,), jnp.float32),
            pltpu.VMEM((K,), jnp.int32),
            pltpu.VMEM((seg,), jnp.float32),
            pltpu.VMEM((seg,), jnp.float32),
            pltpu.VMEM_SHARED((NS * nh,), jnp.float32),
        ],
    )
    def degk(dst, out, hist, dstv, tmp, accv, spm):
        c = lax.axis_index("c")
        s = lax.axis_index("s")

        @pl.loop(0, nh // L)
        def _zero(i):
            hist[pl.ds(pl.multiple_of(i * L, L), L)] = jnp.zeros((L,), jnp.float32)

        ebase = (c * NS + s) * et

        @pl.loop(0, nb)
        def _count(i):
            off = pl.multiple_of(ebase + i * K, K)
            pltpu.sync_copy(dst.at[pl.ds(off, K)], dstv)
            for k in range(K // L):
                idx = dstv[pl.ds(k * L, L)]
                plsc.addupdate_scatter(hist, [idx], jnp.ones((L,), jnp.float32))

        pltpu.sync_copy(hist, spm.at[pl.ds(s * nh, nh)])
        plsc.subcore_barrier()

        pltpu.sync_copy(spm.at[pl.ds(s * seg, seg)], accv)
        for j in range(1, NS):
            pltpu.sync_copy(spm.at[pl.ds(j * nh + s * seg, seg)], tmp)

            @pl.loop(0, seg // L)
            def _red(m):
                o = pl.multiple_of(m * L, L)
                accv[pl.ds(o, L)] = accv[pl.ds(o, L)] + tmp[pl.ds(o, L)]

        pltpu.sync_copy(accv, out.at[pl.ds(c * nh + s * seg, seg)])

    return degk


def _tc_params():
    return pltpu.CompilerParams(vmem_limit_bytes=60 * 1024 * 1024)


def _dot(a, b):
    return lax.dot_general(a, b, (((1,), (0,)), ((), ())),
                           precision=lax.Precision.HIGHEST)


def kernel(x, edge_index, W1, b1, g1, bt1, W2, b2, g2, bt2, W3, b3):
    n, fin = x.shape
    e = edge_index.shape[1]
    h = W1.shape[1]
    fo = W3.shape[1]
    n_acc = -(-n // (NS * ZR)) * (NS * ZR)
    dummy = n
    src = edge_index[0].astype(jnp.int32)
    dst = edge_index[1].astype(jnp.int32)
    zrows = jnp.zeros((ZR, fin), jnp.float32)
    nblk = n // BR
    assert n % BR == 0

    et_deg = -(-e // (NC * NS * K)) * K
    pad = NC * NS * et_deg - e
    dst_deg = jnp.concatenate([dst, jnp.full((pad,), dummy, jnp.int32)])

    et_es = -(-e // (NC * NS * 2 * K)) * (2 * K)
    pad = NC * NS * et_es - e
    src_es = jnp.concatenate([src, jnp.zeros((pad,), jnp.int32)])
    dst_es = jnp.concatenate([dst, jnp.full((pad,), dummy, jnp.int32)])

    et_fs = -(-e // (NS * 2 * K)) * (2 * K)
    pad = NS * et_fs - e
    srcp = jnp.concatenate([src, jnp.zeros((pad,), jnp.int32)])
    dstp = jnp.concatenate([dst, jnp.full((pad,), dummy, jnp.int32)])
    src_fs = jnp.concatenate([srcp, srcp + n])
    dst_fs = jnp.concatenate([dstp, dstp])

    col = lambda w: pl.BlockSpec((1, w), lambda i: (0, 0))
    rows = lambda w: pl.BlockSpec((BR, w), lambda i: (i, 0))
    pair = pl.BlockSpec((NC, BR, fin), lambda i: (0, i, 0))

    degp = _deg_kernel(et_deg, n_acc)(dst_deg).reshape(NC, n_acc, 1)

    def t1(degp_ref, x_ref, dinv_ref, y0_ref):
        deg = degp_ref[0] + degp_ref[1] + 1.0
        dinv = jnp.where(deg > 0.0, lax.rsqrt(deg), 0.0)
        dinv_ref[...] = dinv
        y0_ref[...] = dinv * x_ref[...]

    dinv, y0 = pl.pallas_call(
        t1,
        grid=(nblk,),
        in_specs=[pl.BlockSpec((NC, BR, 1), lambda i: (0, i, 0)), rows(fin)],
        out_specs=[rows(1), rows(fin)],
        out_shape=[jax.ShapeDtypeStruct((n, 1), jnp.float32),
                   jax.ShapeDtypeStruct((n, fin), jnp.float32)],
        compiler_params=_tc_params(),
    )(degp, x)

    s0 = _agg_kernel(n, et_es, n_acc, fin)(y0, src_es, dst_es, zrows)
    s0 = s0.reshape(NC, n_acc, fin)

    def stats_body(z_ref, s_ref, q_ref):
        @pl.when(pl.program_id(0) == 0)
        def _init():
            s_ref[...] = jnp.zeros_like(s_ref)
            q_ref[...] = jnp.zeros_like(q_ref)
        z = z_ref[...]
        s_ref[...] += jnp.sum(z, axis=0, keepdims=True)
        q_ref[...] += jnp.sum(z * z, axis=0, keepdims=True)

    def bn_stats(z):
        w = z.shape[1]
        return pl.pallas_call(
            stats_body,
            grid=(nblk,),
            in_specs=[rows(w)],
            out_specs=[col(w), col(w)],
            out_shape=[jax.ShapeDtypeStruct((1, w), jnp.float32),
                       jax.ShapeDtypeStruct((1, w), jnp.float32)],
            compiler_params=_tc_params(),
        )(z)

    def bn_relu(z, sm, sq, g, bt):
        m = sm * (1.0 / n)
        v = sq * (1.0 / n) - m * m
        return jnp.maximum((z - m) / jnp.sqrt(v + 1e-5) * g + bt, 0.0)

    def t2a(s0_ref, y0_ref, dinv_ref, w1_ref, b1_ref, z1_ref):
        agg0 = dinv_ref[...] * (s0_ref[0] + s0_ref[1] + y0_ref[...])
        z1_ref[...] = _dot(agg0, w1_ref[...]) + b1_ref[...]

    z1 = pl.pallas_call(
        t2a,
        grid=(nblk,),
        in_specs=[pair, rows(fin), rows(1),
                  pl.BlockSpec((fin, h), lambda i: (0, 0)), col(h)],
        out_specs=rows(h),
        out_shape=jax.ShapeDtypeStruct((n, h), jnp.float32),
        compiler_params=_tc_params(),
    )(s0, y0, dinv, W1, b1.reshape(1, h))
    m1, q1 = bn_stats(z1)

    def t2b(z_ref, s_ref, q_ref, g_ref, bt_ref, dinv_ref, w_ref, y2_ref):
        h1 = bn_relu(z_ref[...], s_ref[...], q_ref[...], g_ref[...],
                     bt_ref[...])
        y2 = dinv_ref[...] * _dot(h1, w_ref[...])
        y2_ref[0] = y2[:, :fin]
        y2_ref[1] = y2[:, fin:]

    y2pair = pl.pallas_call(
        t2b,
        grid=(nblk,),
        in_specs=[rows(h), col(h), col(h), col(h), col(h), rows(1),
                  pl.BlockSpec((h, h), lambda i: (0, 0))],
        out_specs=pair,
        out_shape=jax.ShapeDtypeStruct((NC, n, fin), jnp.float32),
        compiler_params=_tc_params(),
    )(z1, m1, q1, g1.reshape(1, h), bt1.reshape(1, h), dinv, W2)

    s2 = _agg_kernel(NC * n, et_fs, n_acc, fin)(
        y2pair.reshape(NC * n, fin), src_fs, dst_fs, zrows)
    s2 = s2.reshape(NC, n_acc, fin)

    def t3a(s2_ref, y2_ref, dinv_ref, b2_ref, z2_ref):
        ssum = jnp.concatenate(
            [s2_ref[0] + y2_ref[0], s2_ref[1] + y2_ref[1]], axis=1)
        z2_ref[...] = dinv_ref[...] * ssum + b2_ref[...]

    z2 = pl.pallas_call(
        t3a,
        grid=(nblk,),
        in_specs=[pair, pair, rows(1), col(h)],
        out_specs=rows(h),
        out_shape=jax.ShapeDtypeStruct((n, h), jnp.float32),
        compiler_params=_tc_params(),
    )(s2, y2pair, dinv, b2.reshape(1, h))
    m2, q2 = bn_stats(z2)

    def t3b(z_ref, s_ref, q_ref, g_ref, bt_ref, dinv_ref, w_ref, y3_ref):
        h2 = bn_relu(z_ref[...], s_ref[...], q_ref[...], g_ref[...],
                     bt_ref[...])
        y3_ref[...] = dinv_ref[...] * _dot(h2, w_ref[...])

    y3 = pl.pallas_call(
        t3b,
        grid=(nblk,),
        in_specs=[rows(h), col(h), col(h), col(h), col(h), rows(1),
                  pl.BlockSpec((h, fo), lambda i: (0, 0))],
        out_specs=rows(fo),
        out_shape=jax.ShapeDtypeStruct((n, fo), jnp.float32),
        compiler_params=_tc_params(),
    )(z2, m2, q2, g2.reshape(1, h), bt2.reshape(1, h), dinv, W3)

    s3 = _agg_kernel(n, et_es, n_acc, fo)(y3, src_es, dst_es, zrows)
    s3 = s3.reshape(NC, n_acc, fo)

    def t4(s3_ref, y3_ref, dinv_ref, b3_ref, o_ref):
        o_ref[...] = dinv_ref[...] * (s3_ref[0] + s3_ref[1] + y3_ref[...]) \
            + b3_ref[...]

    out = pl.pallas_call(
        t4,
        grid=(nblk,),
        in_specs=[pl.BlockSpec((NC, BR, fo), lambda i: (0, i, 0)), rows(fo),
                  rows(1), col(fo)],
        out_specs=rows(fo),
        out_shape=jax.ShapeDtypeStruct((n, fo), jnp.float32),
        compiler_params=_tc_params(),
    )(s3, y3, dinv, b3.reshape(1, fo))
    return out
